# baseline (speedup 1.0000x reference)
import functools

import jax
import jax.numpy as jnp
from jax import lax
from jax.experimental import pallas as pl
from jax.experimental.pallas import tpu as pltpu
from jax.experimental.pallas import tpu_sc as plsc

EMBED_DIM = 32


@functools.lru_cache(maxsize=None)
def _make_gather(b_total: int, chunk: int, table_rows: int):
    info = plsc.get_sparse_core_info()
    num_cores, num_subcores = info.num_cores, info.num_subcores
    num_workers = num_cores * num_subcores
    b_per_w = b_total // num_workers
    assert b_per_w * num_workers == b_total
    n_chunks = b_per_w // chunk
    assert n_chunks * chunk == b_per_w
    mesh = plsc.VectorSubcoreMesh(core_axis_name="c", subcore_axis_name="s")

    @functools.partial(
        pl.kernel,
        mesh=mesh,
        out_type=jax.ShapeDtypeStruct((b_total, EMBED_DIM), jnp.float32),
        scratch_types=[
            pltpu.VMEM_SHARED((table_rows + 1, EMBED_DIM), jnp.float32),
            pltpu.VMEM((4, chunk), jnp.int32),
            [pltpu.VMEM((chunk, EMBED_DIM), jnp.float32) for _ in range(4)],
            [pltpu.SemaphoreType.DMA for _ in range(4)],
            [pltpu.SemaphoreType.DMA for _ in range(4)],
            [pltpu.SemaphoreType.DMA for _ in range(4)],
        ],
        compiler_params=pltpu.CompilerParams(use_tc_tiling_on_sc=False),
    )
    def gather_kernel(table_hbm, idx_hbm, out_2d, table_sh, idx_v, rows, gsem, ssem, isem):
        sid = lax.axis_index("s")
        wid = sid * num_cores + lax.axis_index("c")
        base = wid * b_per_w

        @pl.when(sid == 0)
        def _():
            pltpu.sync_copy(table_hbm, table_sh)

        plsc.subcore_barrier()

        def fire_idx(b, c):
            pltpu.async_copy(
                idx_hbm.at[pl.ds(base + c * chunk, chunk)], idx_v.at[b], isem[b]
            )

        def fire_gather(b):
            pltpu.make_async_copy(
                idx_hbm.at[pl.ds(0, chunk)], idx_v.at[b], isem[b]
            ).wait()
            pltpu.async_copy(table_sh.at[idx_v.at[b]], rows[b], gsem[b])

        def store_of(b, c):
            return pltpu.make_async_copy(
                rows[b],
                out_2d.at[pl.ds(base + c * chunk, chunk)],
                ssem[b],
            )

        for b in range(4):
            fire_idx(b, b)
        for b in range(2):
            fire_gather(b)

        def body(g, carry):
            for b in range(4):
                b2 = (b + 2) % 4
                s = 4 * g + b
                pltpu.make_async_copy(table_sh.at[idx_v.at[b]], rows[b], gsem[b]).wait()
                store_of(b, s).start()

                @pl.when(s + 4 < n_chunks)
                def _():
                    fire_idx(b, s + 4)

                @pl.when(s >= 2)
                def _():
                    store_of(b2, s - 2).wait()

                @pl.when(s + 2 < n_chunks)
                def _():
                    fire_gather(b2)

            return carry

        lax.fori_loop(0, n_chunks // 4, body, 0)

        for c in (n_chunks - 2, n_chunks - 1):
            store_of(c % 4, c).wait()

    return gather_kernel


def kernel(tensor, int_to_feat_matrix, extra_embeddings):
    batch, fields = tensor.shape
    idx = tensor.reshape(-1).astype(jnp.int32)
    table = jnp.concatenate([int_to_feat_matrix, extra_embeddings], axis=0)
    rows = _make_gather(batch * fields, 512, int_to_feat_matrix.shape[0])(table, idx)
    return rows.reshape(batch, fields * EMBED_DIM)


# speedup vs baseline: 1.0062x; 1.0062x over previous
import functools

import jax
import jax.numpy as jnp
from jax import lax
from jax.experimental import pallas as pl
from jax.experimental.pallas import tpu as pltpu
from jax.experimental.pallas import tpu_sc as plsc

EMBED_DIM = 32
NBUF = 8


@functools.lru_cache(maxsize=None)
def _make_gather(b_total: int, chunk: int, table_rows: int):
    info = plsc.get_sparse_core_info()
    num_cores, num_subcores = info.num_cores, info.num_subcores
    num_workers = num_cores * num_subcores
    b_per_w = b_total // num_workers
    assert b_per_w * num_workers == b_total
    n_chunks = b_per_w // chunk
    assert n_chunks * chunk == b_per_w
    assert n_chunks % NBUF == 0
    half = NBUF // 2
    mesh = plsc.VectorSubcoreMesh(core_axis_name="c", subcore_axis_name="s")

    @functools.partial(
        pl.kernel,
        mesh=mesh,
        out_type=jax.ShapeDtypeStruct((b_total, EMBED_DIM), jnp.float32),
        scratch_types=[
            pltpu.VMEM_SHARED((table_rows + 1, EMBED_DIM), jnp.float32),
            pltpu.VMEM((NBUF, chunk), jnp.int32),
            [pltpu.VMEM((chunk, EMBED_DIM), jnp.float32) for _ in range(NBUF)],
            [pltpu.SemaphoreType.DMA for _ in range(NBUF)],
            [pltpu.SemaphoreType.DMA for _ in range(NBUF)],
            [pltpu.SemaphoreType.DMA for _ in range(NBUF)],
        ],
        compiler_params=pltpu.CompilerParams(use_tc_tiling_on_sc=False),
    )
    def gather_kernel(table_hbm, idx_hbm, out_2d, table_sh, idx_v, rows, gsem, ssem, isem):
        sid = lax.axis_index("s")
        wid = sid * num_cores + lax.axis_index("c")
        base = wid * b_per_w

        @pl.when(sid == 0)
        def _():
            pltpu.sync_copy(table_hbm, table_sh)

        plsc.subcore_barrier()

        def fire_idx(b, c):
            pltpu.async_copy(
                idx_hbm.at[pl.ds(base + c * chunk, chunk)], idx_v.at[b], isem[b]
            )

        def fire_gather(b):
            pltpu.make_async_copy(
                idx_hbm.at[pl.ds(0, chunk)], idx_v.at[b], isem[b]
            ).wait()
            pltpu.async_copy(table_sh.at[idx_v.at[b]], rows[b], gsem[b])

        def store_of(b, c):
            return pltpu.make_async_copy(
                rows[b],
                out_2d.at[pl.ds(base + c * chunk, chunk)],
                ssem[b],
            )

        for b in range(NBUF):
            fire_idx(b, b)
        for b in range(half):
            fire_gather(b)

        def body(g, carry):
            for b in range(NBUF):
                b2 = (b + half) % NBUF
                s = NBUF * g + b
                pltpu.make_async_copy(table_sh.at[idx_v.at[b]], rows[b], gsem[b]).wait()
                store_of(b, s).start()

                @pl.when(s + NBUF < n_chunks)
                def _():
                    fire_idx(b, s + NBUF)

                @pl.when(s >= half)
                def _():
                    store_of(b2, s - half).wait()

                @pl.when(s + half < n_chunks)
                def _():
                    fire_gather(b2)

            return carry

        lax.fori_loop(0, n_chunks // NBUF, body, 0)

        for c in range(n_chunks - half, n_chunks):
            store_of(c % NBUF, c).wait()

    return gather_kernel


def kernel(tensor, int_to_feat_matrix, extra_embeddings):
    batch, fields = tensor.shape
    idx = tensor.reshape(-1).astype(jnp.int32)
    table = jnp.concatenate([int_to_feat_matrix, extra_embeddings], axis=0)
    rows = _make_gather(batch * fields, 320, int_to_feat_matrix.shape[0])(table, idx)
    return rows.reshape(batch, fields * EMBED_DIM)


# speedup vs baseline: 1.0091x; 1.0028x over previous
import functools

import jax
import jax.numpy as jnp
from jax import lax
from jax.experimental import pallas as pl
from jax.experimental.pallas import tpu as pltpu
from jax.experimental.pallas import tpu_sc as plsc

EMBED_DIM = 32
NBUF = 8


@functools.lru_cache(maxsize=None)
def _make_gather(b_total: int, chunk: int, table_rows: int):
    info = plsc.get_sparse_core_info()
    num_cores, num_subcores = info.num_cores, info.num_subcores
    num_workers = num_cores * num_subcores
    b_per_w = b_total // num_workers
    assert b_per_w * num_workers == b_total
    n_chunks = b_per_w // chunk
    assert n_chunks * chunk == b_per_w
    assert n_chunks % NBUF == 0
    half = NBUF // 2
    mesh = plsc.VectorSubcoreMesh(core_axis_name="c", subcore_axis_name="s")

    @functools.partial(
        pl.kernel,
        mesh=mesh,
        out_type=jax.ShapeDtypeStruct((b_total, EMBED_DIM), jnp.float32),
        scratch_types=[
            pltpu.VMEM_SHARED((table_rows + 1, EMBED_DIM), jnp.float32),
            pltpu.VMEM((NBUF, chunk), jnp.int32),
            [pltpu.VMEM((chunk, EMBED_DIM), jnp.float32) for _ in range(NBUF)],
            [pltpu.SemaphoreType.DMA for _ in range(NBUF)],
            [pltpu.SemaphoreType.DMA for _ in range(NBUF)],
            [pltpu.SemaphoreType.DMA for _ in range(NBUF)],
        ],
        compiler_params=pltpu.CompilerParams(use_tc_tiling_on_sc=False),
    )
    def gather_kernel(table_hbm, idx_hbm, out_2d, table_sh, idx_v, rows, gsem, ssem, isem):
        sid = lax.axis_index("s")
        wid = sid * num_cores + lax.axis_index("c")
        base = wid * b_per_w

        @pl.when(sid == 0)
        def _():
            pltpu.sync_copy(table_hbm, table_sh)

        plsc.subcore_barrier()

        def fire_idx(b, c):
            pltpu.async_copy(
                idx_hbm.at[pl.ds(base + c * chunk, chunk)], idx_v.at[b], isem[b]
            )

        def fire_gather(b):
            pltpu.make_async_copy(
                idx_hbm.at[pl.ds(0, chunk)], idx_v.at[b], isem[b]
            ).wait()
            pltpu.async_copy(table_sh.at[idx_v.at[b]], rows[b], gsem[b])

        def store_of(b, c):
            return pltpu.make_async_copy(
                rows[b],
                out_2d.at[pl.ds(base + c * chunk, chunk)],
                ssem[b],
            )

        for b in range(NBUF):
            fire_idx(b, b)
        for b in range(half):
            fire_gather(b)

        def body(g, carry):
            for b in range(NBUF):
                b2 = (b + half) % NBUF
                s = NBUF * g + b
                pltpu.make_async_copy(table_sh.at[idx_v.at[b]], rows[b], gsem[b]).wait()
                store_of(b, s).start()

                @pl.when(s + NBUF < n_chunks)
                def _():
                    fire_idx(b, s + NBUF)

                @pl.when(s >= half)
                def _():
                    store_of(b2, s - half).wait()

                @pl.when(s + half < n_chunks)
                def _():
                    fire_gather(b2)

            return carry

        lax.fori_loop(0, n_chunks // NBUF, body, 0)

        for c in range(n_chunks - half, n_chunks):
            store_of(c % NBUF, c).wait()

    return gather_kernel


def kernel(tensor, int_to_feat_matrix, extra_embeddings):
    batch, fields = tensor.shape
    idx = tensor.reshape(-1).astype(jnp.int32)
    table = jnp.concatenate([int_to_feat_matrix, extra_embeddings], axis=0)
    rows = _make_gather(batch * fields, 400, int_to_feat_matrix.shape[0])(table, idx)
    return rows.reshape(batch, fields * EMBED_DIM)
